# baseline (speedup 1.0000x reference)
import functools

import jax
import jax.numpy as jnp
from jax import lax
from jax.experimental import pallas as pl
from jax.experimental.pallas import tpu as pltpu
from jax.experimental.pallas import tpu_sc as plsc

H = 128
LN2 = 0.6931471805599453
NW = 32
GK = 128


def _sc_mesh():
    return plsc.VectorSubcoreMesh(core_axis_name="c", subcore_axis_name="s")


_NBUF = 8


GGK = 64


@functools.lru_cache(maxsize=None)
def _make_gather_rows(v, ep):
    per_w = ep // NW
    n_chunks = per_w // GGK
    assert n_chunks % _NBUF == 0

    @functools.partial(
        pl.kernel,
        out_type=jax.ShapeDtypeStruct((ep, H), jnp.float32),
        mesh=_sc_mesh(),
        scratch_types=[
            pltpu.VMEM((per_w,), jnp.int32),
            [pltpu.VMEM((GGK, H), jnp.float32) for _ in range(_NBUF)],
            [pltpu.SemaphoreType.DMA for _ in range(_NBUF)],
            [pltpu.SemaphoreType.DMA for _ in range(_NBUF)],
        ],
    )
    def k(table, idx, out, idx_v, rows, gsem, wsem):
        wid = lax.axis_index("s") * 2 + lax.axis_index("c")
        base = wid * per_w
        pltpu.sync_copy(idx.at[pl.ds(base, per_w)], idx_v)
        for b in range(_NBUF):
            pltpu.async_copy(table.at[idx_v.at[pl.ds(b * GGK, GGK)]], rows[b],
                             gsem[b])

        def step(j, carry):
            for b in range(_NBUF):
                c = j * _NBUF + b
                pltpu.make_async_copy(table.at[idx_v.at[pl.ds(0, GGK)]],
                                      rows[b], gsem[b]).wait()
                pltpu.async_copy(rows[b], out.at[pl.ds(base + c * GGK, GGK)],
                                 wsem[b])
                p = c + _NBUF

                @pl.when(p < n_chunks)
                def _():
                    pltpu.make_async_copy(
                        rows[b], out.at[pl.ds(base, GGK)], wsem[b]).wait()
                    pltpu.async_copy(
                        table.at[idx_v.at[pl.ds(p * GGK, GGK)]], rows[b],
                        gsem[b])
            return carry

        lax.fori_loop(0, n_chunks // _NBUF, step, 0)
        for b in range(_NBUF):
            pltpu.make_async_copy(rows[b], out.at[pl.ds(base, GGK)],
                                  wsem[b]).wait()

    return k


def _chunks_of(total, size):
    out = []
    off = 0
    while off < total:
        c = min(size, total - off)
        out.append((off, c))
        off += c
    return out


SGK = 64


@functools.lru_cache(maxsize=None)
def _make_scatter_add(e, n, bs):
    per_t = e // 16
    n_pass = -(-n // (2 * bs))
    n_pad = n_pass * 2 * bs
    main_chunks = per_t // SGK
    tail = per_t - main_chunks * SGK
    rows_per_tile = bs // 16
    assert main_chunks % 2 == 0 and tail % 16 == 0

    @functools.partial(
        pl.kernel,
        out_type=jax.ShapeDtypeStruct((n_pad, H), jnp.float32),
        mesh=_sc_mesh(),
        scratch_types=[
            pltpu.VMEM_SHARED((bs + 16, H), jnp.float32),
            [pltpu.VMEM((SGK, H), jnp.float32) for _ in range(2)],
            [pltpu.VMEM((SGK,), jnp.int32) for _ in range(2)],
            pltpu.VMEM((48,), jnp.int32),
            [pltpu.SemaphoreType.DMA for _ in range(2)],
            [pltpu.SemaphoreType.DMA for _ in range(2)],
            [pltpu.SemaphoreType.DMA for _ in range(2)],
        ],
    )
    def k(msg, perm, dst_s, bounds, zeros_hbm, out, shared, rows, dcur,
          bnds_v, sems, dsems, asems):
        ci = lax.axis_index("c")
        sid = lax.axis_index("s")
        dump = bs + lax.iota(jnp.int32, 16)
        lane = lax.iota(jnp.int32, 16)
        pltpu.sync_copy(bounds, bnds_v)

        def bnd(k_):
            return bnds_v[pl.ds(k_, 16)][0]

        for p in range(n_pass):
            kb = p * 2 + ci
            base = kb * bs
            lo = bnd(kb)
            hi = bnd(kb + 1)
            cnt = hi - lo
            st = lo + (cnt * sid) // 16
            en = lo + (cnt * (sid + 1)) // 16
            ast = (st // 8) * 8
            trip = (en - ast + SGK - 1) // SGK
            my0 = sid * rows_per_tile
            for (o, c) in _chunks_of(rows_per_tile, SGK):
                pltpu.sync_copy(zeros_hbm.at[pl.ds(0, c)],
                                shared.at[pl.ds(my0 + o, c)])

            @pl.when(sid == 0)
            def _():
                pltpu.sync_copy(zeros_hbm.at[pl.ds(0, 16)],
                                shared.at[pl.ds(bs, 16)])

            plsc.subcore_barrier()

            def step(c, carry):
                e0 = ast + c * SGK
                pltpu.sync_copy(perm.at[pl.ds(e0, SGK)], dcur[0])
                pltpu.async_copy(msg.at[dcur[0]], rows[0], sems[0])
                pltpu.sync_copy(dst_s.at[pl.ds(e0, SGK)], dcur[1])
                pltpu.make_async_copy(msg.at[dcur[0]], rows[0],
                                      sems[0]).wait()
                for v in range(SGK // 16):
                    dv = dcur[1][pl.ds(v * 16, 16)]
                    t = dv - base
                    e = e0 + v * 16 + lane
                    ok = (t >= 0) & (t < bs) & (e >= st) & (e < en)
                    dcur[1][pl.ds(v * 16, 16)] = jnp.where(ok, t, dump)
                pltpu.sync_copy(rows[0], shared.at[dcur[1]], add=True)
                return carry

            lax.fori_loop(0, trip, step, 0)
            plsc.subcore_barrier()
            for (o, c) in _chunks_of(rows_per_tile, SGK):
                pltpu.sync_copy(shared.at[pl.ds(my0 + o, c)],
                                rows[0].at[pl.ds(0, c)])
                pltpu.sync_copy(rows[0].at[pl.ds(0, c)],
                                out.at[pl.ds(base + my0 + o, c)])
            plsc.subcore_barrier()

    return k


def _seg_bs(n):
    return 8192 if n > 8192 else 4096


def _scatter_aux(dst, n):
    bs = _seg_bs(n)
    shift = bs.bit_length() - 1
    e = dst.shape[0]
    iot = jnp.arange(e, dtype=jnp.int32)
    packed = ((dst.astype(jnp.int32) >> shift) << 18) | iot
    perm = (lax.sort(packed) & ((1 << 18) - 1)).astype(jnp.int32)
    dst_s = dst[perm].astype(jnp.int32)
    nb = -(-n // bs) + 1
    edges = jnp.arange(nb, dtype=jnp.int32) * bs
    bnds = jnp.searchsorted(dst_s, edges).astype(jnp.int32)
    bnds = jnp.concatenate([bnds, jnp.full((48 - nb,), e, jnp.int32)])
    perm_p = jnp.concatenate([perm, jnp.zeros((64,), jnp.int32)])
    dst_sp = jnp.concatenate([dst_s, jnp.full((64,), 1 << 30, jnp.int32)])
    return perm_p, dst_sp, bnds


def _sc_segment_sum(msg, aux, n):
    e = msg.shape[0]
    perm_p, dst_sp, bnds = aux
    bs = _seg_bs(n)
    out = _make_scatter_add(e, n, bs)(msg, perm_p, dst_sp, bnds,
                                      jnp.zeros((GK, H), jnp.float32))
    if out.shape[0] != n:
        out = out[:n]
    return out


def _pad_idx(idx):
    e = idx.shape[0]
    ep = -(-e // (NW * GK)) * (NW * GK)
    if ep != e:
        idx = jnp.concatenate([idx, jnp.zeros((ep - e,), idx.dtype)])
    return idx


def _sc_gather_rows(table, idx_padded, e):
    out = _make_gather_rows(table.shape[0], idx_padded.shape[0])(
        table, idx_padded)
    return out[:e]


def _spk(x):
    return jnp.maximum(x, 0.0) + jnp.log1p(jnp.exp(-jnp.abs(x))) - LN2


def _cutoff_body(r_ref, out_ref):
    r = r_ref[...]
    rc = jnp.clip(r, 0.0, 1.0)
    out_ref[...] = jnp.where(r < 1.0, 0.5 * (jnp.cos(jnp.pi * rc) + 1.0), 0.0)


def _cutoff_tc(r2d, block):
    e = r2d.shape[0]
    return pl.pallas_call(
        _cutoff_body,
        grid=(e // block,),
        in_specs=[pl.BlockSpec((block, 1), lambda i: (i, 0))],
        out_specs=pl.BlockSpec((block, 1), lambda i: (i, 0)),
        out_shape=jax.ShapeDtypeStruct((e, 1), jnp.float32),
    )(r2d)


def _edge_msg_body(r_ref, cut_ref, hsrc_ref, e_ref, w1_ref, b1_ref, w2_ref,
                   b2_ref, out_ref, *, lo, hi):
    r = r_ref[...]
    j = jax.lax.broadcasted_iota(jnp.int32, (1, H), 1).astype(jnp.float32)
    mu = lo + j * ((hi - lo) / (H - 1))
    bf = jnp.exp(-10.0 * (r - mu) ** 2) * cut_ref[...]
    t = _spk(jnp.dot(bf, w1_ref[...], preferred_element_type=jnp.float32)
             + b1_ref[...])
    filt = jnp.dot(t, w2_ref[...], preferred_element_type=jnp.float32) \
        + b2_ref[...]
    out_ref[...] = (hsrc_ref[...] + e_ref[...]) * filt


def _edge_msg_tc(r2d, cut2d, hsrc, e, w1, b1, w2, b2, lo, hi, block):
    n = r2d.shape[0]
    body = functools.partial(_edge_msg_body, lo=lo, hi=hi)
    return pl.pallas_call(
        body,
        grid=(n // block,),
        in_specs=[
            pl.BlockSpec((block, 1), lambda i: (i, 0)),
            pl.BlockSpec((block, 1), lambda i: (i, 0)),
            pl.BlockSpec((block, H), lambda i: (i, 0)),
            pl.BlockSpec((block, H), lambda i: (i, 0)),
            pl.BlockSpec((H, H), lambda i: (0, 0)),
            pl.BlockSpec((1, H), lambda i: (0, 0)),
            pl.BlockSpec((H, H), lambda i: (0, 0)),
            pl.BlockSpec((1, H), lambda i: (0, 0)),
        ],
        out_specs=pl.BlockSpec((block, H), lambda i: (i, 0)),
        out_shape=jax.ShapeDtypeStruct((n, H), jnp.float32),
    )(r2d, cut2d, hsrc, e, w1, b1.reshape(1, H), w2, b2.reshape(1, H))


def _out_update_body(hn_ref, agg_ref, wo_ref, bo_ref, out_ref):
    u = jnp.dot(agg_ref[...], wo_ref[...], preferred_element_type=jnp.float32) \
        + bo_ref[...]
    out_ref[...] = hn_ref[...] + _spk(u)


def _out_update_tc(hn, agg, wo, bo, block):
    n = hn.shape[0]
    return pl.pallas_call(
        _out_update_body,
        grid=(n // block,),
        in_specs=[
            pl.BlockSpec((block, H), lambda i: (i, 0)),
            pl.BlockSpec((block, H), lambda i: (i, 0)),
            pl.BlockSpec((H, H), lambda i: (0, 0)),
            pl.BlockSpec((1, H), lambda i: (0, 0)),
        ],
        out_specs=pl.BlockSpec((block, H), lambda i: (i, 0)),
        out_shape=jax.ShapeDtypeStruct((n, H), jnp.float32),
    )(hn, agg, wo, bo.reshape(1, H))


def _readout1_body(hn_ref, w1_ref, b1_ref, w2_ref, b2_ref, out_ref):
    i = pl.program_id(0)
    t = _spk(jnp.dot(hn_ref[...], w1_ref[...],
                     preferred_element_type=jnp.float32) + b1_ref[...])
    x = jnp.dot(t, w2_ref[...], preferred_element_type=jnp.float32) \
        + b2_ref[...]
    part = jnp.sum(x, axis=0, keepdims=True)

    @pl.when(i == 0)
    def _():
        out_ref[...] = jnp.zeros_like(out_ref)

    out_ref[...] += part


def _readout1_tc(hn, w1, b1, w2, b2, block):
    n = hn.shape[0]
    return pl.pallas_call(
        _readout1_body,
        grid=(n // block,),
        in_specs=[
            pl.BlockSpec((block, H), lambda i: (i, 0)),
            pl.BlockSpec((H, H), lambda i: (0, 0)),
            pl.BlockSpec((1, H), lambda i: (0, 0)),
            pl.BlockSpec((H, H), lambda i: (0, 0)),
            pl.BlockSpec((1, H), lambda i: (0, 0)),
        ],
        out_specs=pl.BlockSpec((1, H), lambda i: (0, 0)),
        out_shape=jax.ShapeDtypeStruct((1, H), jnp.float32),
    )(hn, w1, b1.reshape(1, H), w2, b2.reshape(1, H))


def _readout2_body(m_ref, w1_ref, b1_ref, w2_ref, b2_ref, out_ref):
    t = _spk(jnp.dot(m_ref[...], w1_ref[...],
                     preferred_element_type=jnp.float32) + b1_ref[...])
    out_ref[...] = jnp.dot(t, w2_ref[...],
                           preferred_element_type=jnp.float32) + b2_ref[...]


def _readout2_tc(m, w1, b1, w2, b2):
    return pl.pallas_call(
        _readout2_body,
        in_specs=[
            pl.BlockSpec((1, H), lambda: (0, 0)),
            pl.BlockSpec((H, H), lambda: (0, 0)),
            pl.BlockSpec((1, H), lambda: (0, 0)),
            pl.BlockSpec((H, 1), lambda: (0, 0)),
            pl.BlockSpec((1, 1), lambda: (0, 0)),
        ],
        out_specs=pl.BlockSpec((1, 1), lambda: (0, 0)),
        out_shape=jax.ShapeDtypeStruct((1, 1), jnp.float32),
    )(m, w1, b1.reshape(1, H), w2, b2.reshape(1, 1))


BLK = 2000


def kernel(r_g, r_h, r_i, emb2, emb3, emb4, conv_W1, conv_b1, conv_W2,
           conv_b2, conv_Wo, conv_bo, fc_W1, fc_b1, fc_W2, fc_b2, fc2_W1,
           fc2_b1, fc2_W2, fc2_b2, edge_index_g, edge_index_h, edge_index_i,
           z):
    n_nodes = z.shape[0]
    eg = r_g.shape[0]
    eh = r_h.shape[0]
    ei = r_i.shape[0]

    gs, gd = edge_index_g[0], edge_index_g[1]
    hs, hd = edge_index_h[0], edge_index_h[1]
    isrc, idst = edge_index_i[0], edge_index_i[1]

    r_g2 = r_g.reshape(eg, 1)
    r_h2 = r_h.reshape(eh, 1)
    r_i2 = r_i.reshape(ei, 1)

    cut_g = _cutoff_tc(r_g2, BLK)
    cut_g1 = cut_g.reshape(eg)
    cut_h1 = jnp.minimum(cut_g1[hs], cut_g1[hd])
    cut_h = cut_h1.reshape(eh, 1)
    cut_i = jnp.minimum(cut_h1[isrc], cut_h1[idst]).reshape(ei, 1)

    eq = (z[gs] == z[gd]).astype(jnp.int32)
    he_g = emb2[eq]
    c1 = z[gs[hs]]
    c2 = z[gd[hs]]
    c3 = z[gd[hd]]
    tbits = ((c1 == c2).astype(jnp.int32) + 2 * (c1 == c3).astype(jnp.int32)
             + 4 * (c2 == c3).astype(jnp.int32))
    he_h = emb3[tbits]
    a = hs[isrc]
    b = hd[isrc]
    d = hd[idst]
    q1 = z[gs[a]]
    q2 = z[gd[a]]
    q3 = z[gd[b]]
    q4 = z[gd[d]]
    qbits = ((q1 == q2).astype(jnp.int32) + 2 * (q1 == q3).astype(jnp.int32)
             + 4 * (q1 == q4).astype(jnp.int32)
             + 8 * (q2 == q3).astype(jnp.int32)
             + 16 * (q2 == q4).astype(jnp.int32)
             + 32 * (q3 == q4).astype(jnp.int32))
    he_i = emb4[qbits]

    hn_g = jnp.ones((n_nodes, H), dtype=r_g.dtype)
    hn_h = he_g
    hn_i = he_h

    gsp = _pad_idx(gs)
    hsp = _pad_idx(hs)
    isrcp = _pad_idx(isrc)
    aux_g = _scatter_aux(gd, n_nodes)
    aux_h = _scatter_aux(hd, eg)
    aux_i = _scatter_aux(idst, eh)

    for l in range(3):
        e_h = hn_i
        e_g = hn_h
        msg_g = _edge_msg_tc(r_g2, cut_g, _sc_gather_rows(hn_g, gsp, eg), e_g,
                             conv_W1[l, 0], conv_b1[l, 0], conv_W2[l, 0],
                             conv_b2[l, 0], 0.0, 1.0, BLK)
        agg_g = _sc_segment_sum(msg_g, aux_g, n_nodes)
        hn_g_new = _out_update_tc(hn_g, agg_g, conv_Wo[l, 0], conv_bo[l, 0],
                                  BLK)
        msg_h = _edge_msg_tc(r_h2, cut_h, _sc_gather_rows(hn_h, hsp, eh), e_h,
                             conv_W1[l, 1], conv_b1[l, 1], conv_W2[l, 1],
                             conv_b2[l, 1], -1.0, 1.0, BLK)
        agg_h = _sc_segment_sum(msg_h, aux_h, eg)
        hn_h_new = _out_update_tc(hn_h, agg_h, conv_Wo[l, 1], conv_bo[l, 1],
                                  BLK)
        msg_i = _edge_msg_tc(r_i2, cut_i, _sc_gather_rows(hn_i, isrcp, ei),
                             he_i,
                             conv_W1[l, 2], conv_b1[l, 2], conv_W2[l, 2],
                             conv_b2[l, 2], -1.0, 1.0, BLK)
        agg_i = _sc_segment_sum(msg_i, aux_i, eh)
        hn_i_new = _out_update_tc(hn_i, agg_i, conv_Wo[l, 2], conv_bo[l, 2],
                                  BLK)
        hn_g, hn_h, hn_i = hn_g_new, hn_h_new, hn_i_new

    s = _readout1_tc(hn_g, fc_W1, fc_b1, fc_W2, fc_b2, BLK)
    m = s / jnp.float32(n_nodes)
    y = _readout2_tc(m, fc2_W1, fc2_b1, fc2_W2, fc2_b2)
    return y.reshape(1)


# speedup vs baseline: 1.0119x; 1.0119x over previous
import functools

import jax
import jax.numpy as jnp
from jax import lax
from jax.experimental import pallas as pl
from jax.experimental.pallas import tpu as pltpu
from jax.experimental.pallas import tpu_sc as plsc

H = 128
LN2 = 0.6931471805599453
NW = 32
GK = 128


def _sc_mesh():
    return plsc.VectorSubcoreMesh(core_axis_name="c", subcore_axis_name="s")


_NBUF = 8


GGK = 64


@functools.lru_cache(maxsize=None)
def _make_gather_rows(v, ep):
    per_w = ep // NW
    n_chunks = per_w // GGK
    assert n_chunks % _NBUF == 0

    @functools.partial(
        pl.kernel,
        out_type=jax.ShapeDtypeStruct((ep, H), jnp.float32),
        mesh=_sc_mesh(),
        scratch_types=[
            pltpu.VMEM((per_w,), jnp.int32),
            [pltpu.VMEM((GGK, H), jnp.float32) for _ in range(_NBUF)],
            [pltpu.SemaphoreType.DMA for _ in range(_NBUF)],
            [pltpu.SemaphoreType.DMA for _ in range(_NBUF)],
        ],
    )
    def k(table, idx, out, idx_v, rows, gsem, wsem):
        wid = lax.axis_index("s") * 2 + lax.axis_index("c")
        base = wid * per_w
        pltpu.sync_copy(idx.at[pl.ds(base, per_w)], idx_v)
        for b in range(_NBUF):
            pltpu.async_copy(table.at[idx_v.at[pl.ds(b * GGK, GGK)]], rows[b],
                             gsem[b])

        def step(j, carry):
            for b in range(_NBUF):
                c = j * _NBUF + b
                pltpu.make_async_copy(table.at[idx_v.at[pl.ds(0, GGK)]],
                                      rows[b], gsem[b]).wait()
                pltpu.async_copy(rows[b], out.at[pl.ds(base + c * GGK, GGK)],
                                 wsem[b])
                p = c + _NBUF

                @pl.when(p < n_chunks)
                def _():
                    pltpu.make_async_copy(
                        rows[b], out.at[pl.ds(base, GGK)], wsem[b]).wait()
                    pltpu.async_copy(
                        table.at[idx_v.at[pl.ds(p * GGK, GGK)]], rows[b],
                        gsem[b])
            return carry

        lax.fori_loop(0, n_chunks // _NBUF, step, 0)
        for b in range(_NBUF):
            pltpu.make_async_copy(rows[b], out.at[pl.ds(base, GGK)],
                                  wsem[b]).wait()

    return k


def _chunks_of(total, size):
    out = []
    off = 0
    while off < total:
        c = min(size, total - off)
        out.append((off, c))
        off += c
    return out


SGK = 64


@functools.lru_cache(maxsize=None)
def _make_scatter_add(e, n, bs):
    per_t = e // 16
    n_pass = -(-n // (2 * bs))
    n_pad = n_pass * 2 * bs
    main_chunks = per_t // SGK
    tail = per_t - main_chunks * SGK
    rows_per_tile = bs // 16
    assert main_chunks % 2 == 0 and tail % 16 == 0

    @functools.partial(
        pl.kernel,
        out_type=jax.ShapeDtypeStruct((n_pad, H), jnp.float32),
        mesh=_sc_mesh(),
        scratch_types=[
            pltpu.VMEM_SHARED((bs + 16, H), jnp.float32),
            [pltpu.VMEM((SGK, H), jnp.float32) for _ in range(2)],
            [pltpu.VMEM((SGK,), jnp.int32) for _ in range(2)],
            pltpu.VMEM((48,), jnp.int32),
            [pltpu.SemaphoreType.DMA for _ in range(2)],
            [pltpu.SemaphoreType.DMA for _ in range(2)],
            [pltpu.SemaphoreType.DMA for _ in range(2)],
        ],
    )
    def k(msg, perm, dst_s, bounds, zeros_hbm, out, shared, rows, dcur,
          bnds_v, sems, dsems, asems):
        ci = lax.axis_index("c")
        sid = lax.axis_index("s")
        dump = bs + lax.iota(jnp.int32, 16)
        lane = lax.iota(jnp.int32, 16)
        pltpu.sync_copy(bounds, bnds_v)

        def bnd(k_):
            return bnds_v[pl.ds(k_, 16)][0]

        for p in range(n_pass):
            kb = p * 2 + ci
            base = kb * bs
            lo = bnd(kb)
            hi = bnd(kb + 1)
            cnt = hi - lo
            st = lo + (cnt * sid) // 16
            en = lo + (cnt * (sid + 1)) // 16
            ast = (st // 8) * 8
            trip = (en - ast + SGK - 1) // SGK
            my0 = sid * rows_per_tile
            for (o, c) in _chunks_of(rows_per_tile, SGK):
                pltpu.sync_copy(zeros_hbm.at[pl.ds(0, c)],
                                shared.at[pl.ds(my0 + o, c)])

            @pl.when(sid == 0)
            def _():
                pltpu.sync_copy(zeros_hbm.at[pl.ds(0, 16)],
                                shared.at[pl.ds(bs, 16)])

            plsc.subcore_barrier()

            def step(c, carry):
                e0 = ast + c * SGK
                pltpu.sync_copy(perm.at[pl.ds(e0, SGK)], dcur[0])
                pltpu.async_copy(msg.at[dcur[0]], rows[0], sems[0])
                pltpu.sync_copy(dst_s.at[pl.ds(e0, SGK)], dcur[1])
                pltpu.make_async_copy(msg.at[dcur[0]], rows[0],
                                      sems[0]).wait()
                for v in range(SGK // 16):
                    dv = dcur[1][pl.ds(v * 16, 16)]
                    t = dv - base
                    e = e0 + v * 16 + lane
                    ok = (t >= 0) & (t < bs) & (e >= st) & (e < en)
                    dcur[1][pl.ds(v * 16, 16)] = jnp.where(ok, t, dump)
                pltpu.sync_copy(rows[0], shared.at[dcur[1]], add=True)
                return carry

            lax.fori_loop(0, trip, step, 0)
            plsc.subcore_barrier()
            for (o, c) in _chunks_of(rows_per_tile, SGK):
                pltpu.sync_copy(shared.at[pl.ds(my0 + o, c)],
                                rows[0].at[pl.ds(0, c)])
                pltpu.sync_copy(rows[0].at[pl.ds(0, c)],
                                out.at[pl.ds(base + my0 + o, c)])
            plsc.subcore_barrier()

    return k


_BS_BIG = 14208


def _seg_bs(n):
    return _BS_BIG if n > 2 * _BS_BIG else (-(-n // 256) * 128)


def _scatter_aux(dst, n):
    bs = _seg_bs(n)
    perm = jnp.argsort(dst).astype(jnp.int32)
    dst_s = dst[perm].astype(jnp.int32)
    edges = jnp.arange(13, dtype=jnp.int32) * bs
    bnds = jnp.searchsorted(dst_s, edges).astype(jnp.int32)
    bnds = jnp.concatenate([bnds, jnp.full((35,), dst.shape[0], jnp.int32)])
    perm_p = jnp.concatenate([perm, jnp.zeros((64,), jnp.int32)])
    dst_sp = jnp.concatenate([dst_s, jnp.full((64,), 1 << 30, jnp.int32)])
    return perm_p, dst_sp, bnds


def _sc_segment_sum(msg, aux, n):
    e = msg.shape[0]
    perm_p, dst_sp, bnds = aux
    bs = _seg_bs(n)
    out = _make_scatter_add(e, n, bs)(msg, perm_p, dst_sp, bnds,
                                      jnp.zeros((GK, H), jnp.float32))
    if out.shape[0] != n:
        out = out[:n]
    return out


def _pad_idx(idx):
    e = idx.shape[0]
    ep = -(-e // (NW * GK)) * (NW * GK)
    if ep != e:
        idx = jnp.concatenate([idx, jnp.zeros((ep - e,), idx.dtype)])
    return idx


def _sc_gather_rows(table, idx_padded, e):
    out = _make_gather_rows(table.shape[0], idx_padded.shape[0])(
        table, idx_padded)
    return out[:e]


def _spk(x):
    return jnp.maximum(x, 0.0) + jnp.log1p(jnp.exp(-jnp.abs(x))) - LN2


def _cutoff_body(r_ref, out_ref):
    r = r_ref[...]
    rc = jnp.clip(r, 0.0, 1.0)
    out_ref[...] = jnp.where(r < 1.0, 0.5 * (jnp.cos(jnp.pi * rc) + 1.0), 0.0)


def _cutoff_tc(r2d, block):
    e = r2d.shape[0]
    return pl.pallas_call(
        _cutoff_body,
        grid=(e // block,),
        in_specs=[pl.BlockSpec((block, 1), lambda i: (i, 0))],
        out_specs=pl.BlockSpec((block, 1), lambda i: (i, 0)),
        out_shape=jax.ShapeDtypeStruct((e, 1), jnp.float32),
    )(r2d)


def _edge_msg_body(r_ref, cut_ref, hsrc_ref, e_ref, w1_ref, b1_ref, w2_ref,
                   b2_ref, out_ref, *, lo, hi):
    r = r_ref[...]
    j = jax.lax.broadcasted_iota(jnp.int32, (1, H), 1).astype(jnp.float32)
    mu = lo + j * ((hi - lo) / (H - 1))
    bf = jnp.exp(-10.0 * (r - mu) ** 2) * cut_ref[...]
    t = _spk(jnp.dot(bf, w1_ref[...], preferred_element_type=jnp.float32)
             + b1_ref[...])
    filt = jnp.dot(t, w2_ref[...], preferred_element_type=jnp.float32) \
        + b2_ref[...]
    out_ref[...] = (hsrc_ref[...] + e_ref[...]) * filt


def _edge_msg_tc(r2d, cut2d, hsrc, e, w1, b1, w2, b2, lo, hi, block):
    n = r2d.shape[0]
    body = functools.partial(_edge_msg_body, lo=lo, hi=hi)
    return pl.pallas_call(
        body,
        grid=(n // block,),
        in_specs=[
            pl.BlockSpec((block, 1), lambda i: (i, 0)),
            pl.BlockSpec((block, 1), lambda i: (i, 0)),
            pl.BlockSpec((block, H), lambda i: (i, 0)),
            pl.BlockSpec((block, H), lambda i: (i, 0)),
            pl.BlockSpec((H, H), lambda i: (0, 0)),
            pl.BlockSpec((1, H), lambda i: (0, 0)),
            pl.BlockSpec((H, H), lambda i: (0, 0)),
            pl.BlockSpec((1, H), lambda i: (0, 0)),
        ],
        out_specs=pl.BlockSpec((block, H), lambda i: (i, 0)),
        out_shape=jax.ShapeDtypeStruct((n, H), jnp.float32),
    )(r2d, cut2d, hsrc, e, w1, b1.reshape(1, H), w2, b2.reshape(1, H))


def _out_update_body(hn_ref, agg_ref, wo_ref, bo_ref, out_ref):
    u = jnp.dot(agg_ref[...], wo_ref[...], preferred_element_type=jnp.float32) \
        + bo_ref[...]
    out_ref[...] = hn_ref[...] + _spk(u)


def _out_update_tc(hn, agg, wo, bo, block):
    n = hn.shape[0]
    return pl.pallas_call(
        _out_update_body,
        grid=(n // block,),
        in_specs=[
            pl.BlockSpec((block, H), lambda i: (i, 0)),
            pl.BlockSpec((block, H), lambda i: (i, 0)),
            pl.BlockSpec((H, H), lambda i: (0, 0)),
            pl.BlockSpec((1, H), lambda i: (0, 0)),
        ],
        out_specs=pl.BlockSpec((block, H), lambda i: (i, 0)),
        out_shape=jax.ShapeDtypeStruct((n, H), jnp.float32),
    )(hn, agg, wo, bo.reshape(1, H))


def _readout1_body(hn_ref, w1_ref, b1_ref, w2_ref, b2_ref, out_ref):
    i = pl.program_id(0)
    t = _spk(jnp.dot(hn_ref[...], w1_ref[...],
                     preferred_element_type=jnp.float32) + b1_ref[...])
    x = jnp.dot(t, w2_ref[...], preferred_element_type=jnp.float32) \
        + b2_ref[...]
    part = jnp.sum(x, axis=0, keepdims=True)

    @pl.when(i == 0)
    def _():
        out_ref[...] = jnp.zeros_like(out_ref)

    out_ref[...] += part


def _readout1_tc(hn, w1, b1, w2, b2, block):
    n = hn.shape[0]
    return pl.pallas_call(
        _readout1_body,
        grid=(n // block,),
        in_specs=[
            pl.BlockSpec((block, H), lambda i: (i, 0)),
            pl.BlockSpec((H, H), lambda i: (0, 0)),
            pl.BlockSpec((1, H), lambda i: (0, 0)),
            pl.BlockSpec((H, H), lambda i: (0, 0)),
            pl.BlockSpec((1, H), lambda i: (0, 0)),
        ],
        out_specs=pl.BlockSpec((1, H), lambda i: (0, 0)),
        out_shape=jax.ShapeDtypeStruct((1, H), jnp.float32),
    )(hn, w1, b1.reshape(1, H), w2, b2.reshape(1, H))


def _readout2_body(m_ref, w1_ref, b1_ref, w2_ref, b2_ref, out_ref):
    t = _spk(jnp.dot(m_ref[...], w1_ref[...],
                     preferred_element_type=jnp.float32) + b1_ref[...])
    out_ref[...] = jnp.dot(t, w2_ref[...],
                           preferred_element_type=jnp.float32) + b2_ref[...]


def _readout2_tc(m, w1, b1, w2, b2):
    return pl.pallas_call(
        _readout2_body,
        in_specs=[
            pl.BlockSpec((1, H), lambda: (0, 0)),
            pl.BlockSpec((H, H), lambda: (0, 0)),
            pl.BlockSpec((1, H), lambda: (0, 0)),
            pl.BlockSpec((H, 1), lambda: (0, 0)),
            pl.BlockSpec((1, 1), lambda: (0, 0)),
        ],
        out_specs=pl.BlockSpec((1, 1), lambda: (0, 0)),
        out_shape=jax.ShapeDtypeStruct((1, 1), jnp.float32),
    )(m, w1, b1.reshape(1, H), w2, b2.reshape(1, 1))


BLK = 2000


def kernel(r_g, r_h, r_i, emb2, emb3, emb4, conv_W1, conv_b1, conv_W2,
           conv_b2, conv_Wo, conv_bo, fc_W1, fc_b1, fc_W2, fc_b2, fc2_W1,
           fc2_b1, fc2_W2, fc2_b2, edge_index_g, edge_index_h, edge_index_i,
           z):
    n_nodes = z.shape[0]
    eg = r_g.shape[0]
    eh = r_h.shape[0]
    ei = r_i.shape[0]

    gs, gd = edge_index_g[0], edge_index_g[1]
    hs, hd = edge_index_h[0], edge_index_h[1]
    isrc, idst = edge_index_i[0], edge_index_i[1]

    r_g2 = r_g.reshape(eg, 1)
    r_h2 = r_h.reshape(eh, 1)
    r_i2 = r_i.reshape(ei, 1)

    cut_g = _cutoff_tc(r_g2, BLK)
    cut_g1 = cut_g.reshape(eg)
    cut_h1 = jnp.minimum(cut_g1[hs], cut_g1[hd])
    cut_h = cut_h1.reshape(eh, 1)
    cut_i = jnp.minimum(cut_h1[isrc], cut_h1[idst]).reshape(ei, 1)

    eq = (z[gs] == z[gd]).astype(jnp.int32)
    he_g = emb2[eq]
    c1 = z[gs[hs]]
    c2 = z[gd[hs]]
    c3 = z[gd[hd]]
    tbits = ((c1 == c2).astype(jnp.int32) + 2 * (c1 == c3).astype(jnp.int32)
             + 4 * (c2 == c3).astype(jnp.int32))
    he_h = emb3[tbits]
    a = hs[isrc]
    b = hd[isrc]
    d = hd[idst]
    q1 = z[gs[a]]
    q2 = z[gd[a]]
    q3 = z[gd[b]]
    q4 = z[gd[d]]
    qbits = ((q1 == q2).astype(jnp.int32) + 2 * (q1 == q3).astype(jnp.int32)
             + 4 * (q1 == q4).astype(jnp.int32)
             + 8 * (q2 == q3).astype(jnp.int32)
             + 16 * (q2 == q4).astype(jnp.int32)
             + 32 * (q3 == q4).astype(jnp.int32))
    he_i = emb4[qbits]

    hn_g = jnp.ones((n_nodes, H), dtype=r_g.dtype)
    hn_h = he_g
    hn_i = he_h

    gsp = _pad_idx(gs)
    hsp = _pad_idx(hs)
    isrcp = _pad_idx(isrc)
    aux_g = _scatter_aux(gd, n_nodes)
    aux_h = _scatter_aux(hd, eg)
    aux_i = _scatter_aux(idst, eh)

    for l in range(3):
        e_h = hn_i
        e_g = hn_h
        msg_g = _edge_msg_tc(r_g2, cut_g, _sc_gather_rows(hn_g, gsp, eg), e_g,
                             conv_W1[l, 0], conv_b1[l, 0], conv_W2[l, 0],
                             conv_b2[l, 0], 0.0, 1.0, BLK)
        agg_g = _sc_segment_sum(msg_g, aux_g, n_nodes)
        hn_g_new = _out_update_tc(hn_g, agg_g, conv_Wo[l, 0], conv_bo[l, 0],
                                  BLK)
        msg_h = _edge_msg_tc(r_h2, cut_h, _sc_gather_rows(hn_h, hsp, eh), e_h,
                             conv_W1[l, 1], conv_b1[l, 1], conv_W2[l, 1],
                             conv_b2[l, 1], -1.0, 1.0, BLK)
        agg_h = _sc_segment_sum(msg_h, aux_h, eg)
        hn_h_new = _out_update_tc(hn_h, agg_h, conv_Wo[l, 1], conv_bo[l, 1],
                                  BLK)
        msg_i = _edge_msg_tc(r_i2, cut_i, _sc_gather_rows(hn_i, isrcp, ei),
                             he_i,
                             conv_W1[l, 2], conv_b1[l, 2], conv_W2[l, 2],
                             conv_b2[l, 2], -1.0, 1.0, BLK)
        agg_i = _sc_segment_sum(msg_i, aux_i, eh)
        hn_i_new = _out_update_tc(hn_i, agg_i, conv_Wo[l, 2], conv_bo[l, 2],
                                  BLK)
        hn_g, hn_h, hn_i = hn_g_new, hn_h_new, hn_i_new

    s = _readout1_tc(hn_g, fc_W1, fc_b1, fc_W2, fc_b2, BLK)
    m = s / jnp.float32(n_nodes)
    y = _readout2_tc(m, fc2_W1, fc2_b1, fc2_W2, fc2_b2)
    return y.reshape(1)
